# baseline (speedup 1.0000x reference)
import functools

import jax
import jax.numpy as jnp
from jax import lax
from jax.experimental import pallas as pl
from jax.experimental.pallas import tpu as pltpu
from jax.experimental.pallas import tpu_sc as plsc

N = 10000
D = 128
R = 4
OUT = 100
OUT_PAD = 128
B = 1024
E = 320000

NC = 2
NS = 16
NW = NC * NS
CHUNK = 128
CH = 80
AC = 64
ACH = 160
AKB = 32
NBUF = 5
LEAD = 3
E_PAD = NW * CH * CHUNK
N_PAD = 10112
ROWS_PER_SUB = N_PAD // NS
NBLK = 10
BN = N // NBLK
GB = 2 * B // NW


def _transform_body(h_ref, w_ref, o_ref):
    o_ref[...] = jnp.dot(h_ref[...], w_ref[0],
                         preferred_element_type=jnp.float32)


def _transform(h, w, dout):
    return pl.pallas_call(
        _transform_body,
        grid=(R, NBLK),
        in_specs=[
            pl.BlockSpec((BN, D), lambda r, i: (i, 0)),
            pl.BlockSpec((1, D, dout), lambda r, i: (r, 0, 0)),
        ],
        out_specs=pl.BlockSpec((BN, dout), lambda r, i: (r * NBLK + i, 0)),
        out_shape=jax.ShapeDtypeStruct((R * N, dout), jnp.float32),
    )(h, w)


def _combine_body(relu, p_ref, degp_ref, h_ref, ws_ref, o_ref):
    deg = degp_ref[0, :, 0:1] + degp_ref[1, :, 0:1]
    agg = (p_ref[0] + p_ref[1]) / jnp.maximum(deg, 1.0)
    out = agg + jnp.dot(h_ref[...], ws_ref[...],
                        preferred_element_type=jnp.float32)
    if relu:
        out = jnp.maximum(out, 0.0)
    o_ref[...] = out


def _combine_transform_body(p_ref, degp_ref, h_ref, ws_ref, wn_ref,
                            hn_ref, tn_ref):
    deg = degp_ref[0, :, 0:1] + degp_ref[1, :, 0:1]
    agg = (p_ref[0] + p_ref[1]) / jnp.maximum(deg, 1.0)
    hn = jnp.maximum(
        agg + jnp.dot(h_ref[...], ws_ref[...],
                      preferred_element_type=jnp.float32), 0.0)
    hn_ref[...] = hn
    for r in range(R):
        tn_ref[r] = jnp.dot(hn, wn_ref[r],
                            preferred_element_type=jnp.float32)


def _combine_transform(p, degp, h, ws, wn):
    return pl.pallas_call(
        _combine_transform_body,
        grid=(NBLK,),
        in_specs=[
            pl.BlockSpec((2, BN, D), lambda i: (0, i, 0)),
            pl.BlockSpec((2, BN, D), lambda i: (0, i, 0)),
            pl.BlockSpec((BN, D), lambda i: (i, 0)),
            pl.BlockSpec((D, D), lambda i: (0, 0)),
            pl.BlockSpec((R, D, D), lambda i: (0, 0, 0)),
        ],
        out_specs=[
            pl.BlockSpec((BN, D), lambda i: (i, 0)),
            pl.BlockSpec((R, BN, D), lambda i: (0, i, 0)),
        ],
        out_shape=[
            jax.ShapeDtypeStruct((N, D), jnp.float32),
            jax.ShapeDtypeStruct((R, N, D), jnp.float32),
        ],
    )(p, degp, h, ws, wn)


def _combine(p, degp, h, ws, relu, dout):
    return pl.pallas_call(
        functools.partial(_combine_body, relu),
        grid=(NBLK,),
        in_specs=[
            pl.BlockSpec((2, BN, dout), lambda i: (0, i, 0)),
            pl.BlockSpec((2, BN, D), lambda i: (0, i, 0)),
            pl.BlockSpec((BN, D), lambda i: (i, 0)),
            pl.BlockSpec((D, dout), lambda i: (0, 0)),
        ],
        out_specs=pl.BlockSpec((BN, dout), lambda i: (i, 0)),
        out_shape=jax.ShapeDtypeStruct((N, dout), jnp.float32),
    )(p, degp, h, ws)


def _rix_body(s_ref, e_ref, o_ref):
    o_ref[...] = e_ref[...] * N + s_ref[...]


def _rix(srcf, etf):
    rows = E_PAD // 128
    return pl.pallas_call(
        _rix_body,
        grid=(8,),
        in_specs=[pl.BlockSpec((rows // 8, 128), lambda i: (i, 0))] * 2,
        out_specs=pl.BlockSpec((rows // 8, 128), lambda i: (i, 0)),
        out_shape=jax.ShapeDtypeStruct((rows, 128), jnp.int32),
    )(srcf.reshape(rows, 128), etf.reshape(rows, 128))


_MESH = dict(core_axis_name="c", subcore_axis_name="s")


def _sc_aggregate(t, rixw, dstw, zagg):
    @functools.partial(
        pl.kernel,
        out_type=jax.ShapeDtypeStruct((NC, N_PAD, D), jnp.float32),
        mesh=plsc.VectorSubcoreMesh(**_MESH),
        scratch_types=[
            pltpu.VMEM((AKB, AC), jnp.int32),
            pltpu.VMEM((AKB, AC), jnp.int32),
            pltpu.VMEM((AC, D), jnp.float32),
            pltpu.VMEM((AC, D), jnp.float32),
            pltpu.VMEM((AC, D), jnp.float32),
            pltpu.VMEM((AC, D), jnp.float32),
            pltpu.VMEM((AC, D), jnp.float32),
            pltpu.VMEM_SHARED((N_PAD, D), jnp.float32),
            pltpu.SemaphoreType.DMA,
            pltpu.SemaphoreType.DMA,
        ],
    )
    def k(t_hbm, rix_hbm, dst_hbm, z_hbm, p_hbm,
          rix_v, dst_v, m0, m1, m2, m3, m4, agg_sh, semg, sems):
        c = lax.axis_index("c")
        s = lax.axis_index("s")
        wid = s * NC + c
        m = (m0, m1, m2, m3, m4)

        lo = s * ROWS_PER_SUB
        pltpu.sync_copy(z_hbm.at[pl.ds(lo, ROWS_PER_SUB)],
                        agg_sh.at[pl.ds(lo, ROWS_PER_SUB)])
        plsc.subcore_barrier()

        @pl.loop(0, ACH, step=AKB)
        def _(blk):
            pltpu.sync_copy(rix_hbm.at[wid, pl.ds(blk, AKB)], rix_v)
            pltpu.sync_copy(dst_hbm.at[wid, pl.ds(blk, AKB)], dst_v)

            g = [None] * NBUF
            sc = [None] * NBUF
            for ch in range(AKB + LEAD):
                if ch < AKB:
                    b = ch % NBUF
                    if sc[b] is not None:
                        sc[b].wait()
                        sc[b] = None
                    g[b] = pltpu.async_copy(
                        t_hbm.at[rix_v.at[ch]], m[b], semg)
                if ch >= LEAD:
                    b = (ch - LEAD) % NBUF
                    g[b].wait()
                    sc[b] = pltpu.async_copy(
                        m[b], agg_sh.at[dst_v.at[ch - LEAD]], sems, add=True)
            for b in range(NBUF):
                if sc[b] is not None:
                    sc[b].wait()

        plsc.subcore_barrier()
        pltpu.sync_copy(agg_sh.at[pl.ds(lo, ROWS_PER_SUB)],
                        p_hbm.at[c, pl.ds(lo, ROWS_PER_SUB)])

    return k(t, rixw, dstw, zagg)


def _sc_degree(dstw, ones, zdeg):
    @functools.partial(
        pl.kernel,
        out_type=jax.ShapeDtypeStruct((NC, N_PAD, D), jnp.float32),
        mesh=plsc.VectorSubcoreMesh(**_MESH),
        scratch_types=[
            pltpu.VMEM((CH, CHUNK), jnp.int32),
            pltpu.VMEM((CHUNK, D), jnp.float32),
            pltpu.VMEM_SHARED((N_PAD, D), jnp.float32),
            pltpu.SemaphoreType.DMA,
        ],
    )
    def k(dst_hbm, ones_hbm, zd_hbm, degp_hbm, dst_v, ones_v, deg_sh, sems):
        c = lax.axis_index("c")
        s = lax.axis_index("s")
        wid = s * NC + c

        pltpu.sync_copy(ones_hbm, ones_v)
        lo = s * ROWS_PER_SUB
        pltpu.sync_copy(zd_hbm.at[pl.ds(lo, ROWS_PER_SUB)],
                        deg_sh.at[pl.ds(lo, ROWS_PER_SUB)])
        plsc.subcore_barrier()

        pltpu.sync_copy(dst_hbm.at[wid], dst_v)
        descs = [pltpu.async_copy(
                     ones_v, deg_sh.at[dst_v.at[ch]], sems, add=True)
                 for ch in range(CH)]
        for d in descs:
            d.wait()

        plsc.subcore_barrier()
        pltpu.sync_copy(deg_sh.at[pl.ds(lo, ROWS_PER_SUB)],
                        degp_hbm.at[c, pl.ds(lo, ROWS_PER_SUB)])

    return k(dstw, ones, zdeg)


def _sc_gather(h3, idx):
    idxw = idx.reshape(NW, GB)

    @functools.partial(
        pl.kernel,
        out_type=jax.ShapeDtypeStruct((2 * B, OUT_PAD), jnp.float32),
        mesh=plsc.VectorSubcoreMesh(**_MESH),
        scratch_types=[
            pltpu.VMEM((GB,), jnp.int32),
            pltpu.VMEM((GB, OUT_PAD), jnp.float32),
            pltpu.SemaphoreType.DMA,
        ],
    )
    def k(h_hbm, i_hbm, o_hbm, idx_v, rows_v, sem):
        c = lax.axis_index("c")
        s = lax.axis_index("s")
        wid = s * NC + c
        pltpu.sync_copy(i_hbm.at[wid], idx_v)
        pltpu.async_copy(h_hbm.at[idx_v], rows_v, sem).wait()
        pltpu.sync_copy(rows_v, o_hbm.at[pl.ds(wid * GB, GB)])

    return k(h3, idxw)


def kernel(feature, W1, Ws1, W2, Ws2, W3, Ws3,
           edge_index, edge_type, head_idx, tail_idx):
    src = edge_index[0]
    dst = edge_index[1]
    pad = E_PAD - E
    srcf = jnp.concatenate([src, jnp.zeros((pad,), jnp.int32)])
    dstf = jnp.concatenate([dst, jnp.full((pad,), N, jnp.int32)])
    etf = jnp.concatenate([edge_type, jnp.zeros((pad,), jnp.int32)])
    dstw = dstf.reshape(NW, CH, CHUNK)
    dsta = dstf.reshape(NW, ACH, AC)
    rixw = _rix(srcf, etf).reshape(NW, ACH, AC)

    zagg = jnp.zeros((N_PAD, D), jnp.float32)
    ones = jnp.ones((CHUNK, D), jnp.float32)

    w3p = jnp.pad(W3, ((0, 0), (0, 0), (0, OUT_PAD - OUT)))
    ws3p = jnp.pad(Ws3, ((0, 0), (0, OUT_PAD - OUT)))

    degp = _sc_degree(dstw, ones, zagg)

    t1 = _transform(feature, W1, D)
    p1 = _sc_aggregate(t1, rixw, dsta, zagg)
    h1, t2 = _combine_transform(p1, degp, feature, Ws1, W2)

    p2 = _sc_aggregate(t2.reshape(R * N, D), rixw, dsta, zagg)
    h2, t3 = _combine_transform(p2, degp, h1, Ws2, w3p)

    p3 = _sc_aggregate(t3.reshape(R * N, D), rixw, dsta, zagg)
    h3 = _combine(p3, degp, h2, ws3p, False, OUT_PAD)

    ht = _sc_gather(h3, jnp.concatenate([head_idx, tail_idx]))
    return ht[:B, :OUT], ht[B:, :OUT]


# speedup vs baseline: 1.2453x; 1.2453x over previous
import functools

import jax
import jax.numpy as jnp
from jax import lax
from jax.experimental import pallas as pl
from jax.experimental.pallas import tpu as pltpu
from jax.experimental.pallas import tpu_sc as plsc

N = 10000
D = 128
R = 4
OUT = 100
OUT_PAD = 128
B = 1024
E = 320000

NC = 2
NS = 16
NW = NC * NS
CHUNK = 128
CH = 80
AC = 64
ACH = 160
AKB = 32
NBUF = 5
LEAD = 3
E_PAD = NW * CH * CHUNK
N_PAD = 10112
ROWS_PER_SUB = N_PAD // NS
NBLK = 10
BN = N // NBLK
GB = 2 * B // NW


def _transform_body(h_ref, w_ref, o_ref):
    o_ref[...] = jnp.dot(h_ref[...], w_ref[0],
                         preferred_element_type=jnp.float32)


def _transform(h, w, dout):
    return pl.pallas_call(
        _transform_body,
        grid=(R, NBLK),
        in_specs=[
            pl.BlockSpec((BN, D), lambda r, i: (i, 0)),
            pl.BlockSpec((1, D, dout), lambda r, i: (r, 0, 0)),
        ],
        out_specs=pl.BlockSpec((BN, dout), lambda r, i: (r * NBLK + i, 0)),
        out_shape=jax.ShapeDtypeStruct((R * N, dout), jnp.float32),
    )(h, w)


def _combine_body(relu, p_ref, degp_ref, h_ref, ws_ref, o_ref):
    deg = degp_ref[0, :, 0:1] + degp_ref[1, :, 0:1]
    agg = (p_ref[0] + p_ref[1]) / jnp.maximum(deg, 1.0)
    out = agg + jnp.dot(h_ref[...], ws_ref[...],
                        preferred_element_type=jnp.float32)
    if relu:
        out = jnp.maximum(out, 0.0)
    o_ref[...] = out


def _combine(p, degp, h, ws, relu, dout):
    return pl.pallas_call(
        functools.partial(_combine_body, relu),
        grid=(NBLK,),
        in_specs=[
            pl.BlockSpec((2, BN, dout), lambda i: (0, i, 0)),
            pl.BlockSpec((2, BN, D), lambda i: (0, i, 0)),
            pl.BlockSpec((BN, D), lambda i: (i, 0)),
            pl.BlockSpec((D, dout), lambda i: (0, 0)),
        ],
        out_specs=pl.BlockSpec((BN, dout), lambda i: (i, 0)),
        out_shape=jax.ShapeDtypeStruct((N, dout), jnp.float32),
    )(p, degp, h, ws)


def _rix_body(s_ref, e_ref, o_ref):
    o_ref[...] = e_ref[...] * N + s_ref[...]


def _rix(srcf, etf):
    rows = E_PAD // 128
    return pl.pallas_call(
        _rix_body,
        grid=(8,),
        in_specs=[pl.BlockSpec((rows // 8, 128), lambda i: (i, 0))] * 2,
        out_specs=pl.BlockSpec((rows // 8, 128), lambda i: (i, 0)),
        out_shape=jax.ShapeDtypeStruct((rows, 128), jnp.int32),
    )(srcf.reshape(rows, 128), etf.reshape(rows, 128))


_MESH = dict(core_axis_name="c", subcore_axis_name="s")


def _sc_aggregate(t, rixw, dstw, zagg):
    @functools.partial(
        pl.kernel,
        out_type=jax.ShapeDtypeStruct((NC, N_PAD, D), jnp.float32),
        mesh=plsc.VectorSubcoreMesh(**_MESH),
        scratch_types=[
            pltpu.VMEM((AKB, AC), jnp.int32),
            pltpu.VMEM((AKB, AC), jnp.int32),
            pltpu.VMEM((AC, D), jnp.float32),
            pltpu.VMEM((AC, D), jnp.float32),
            pltpu.VMEM((AC, D), jnp.float32),
            pltpu.VMEM((AC, D), jnp.float32),
            pltpu.VMEM((AC, D), jnp.float32),
            pltpu.VMEM_SHARED((N_PAD, D), jnp.float32),
            pltpu.SemaphoreType.DMA,
            pltpu.SemaphoreType.DMA,
        ],
    )
    def k(t_hbm, rix_hbm, dst_hbm, z_hbm, p_hbm,
          rix_v, dst_v, m0, m1, m2, m3, m4, agg_sh, semg, sems):
        c = lax.axis_index("c")
        s = lax.axis_index("s")
        wid = s * NC + c
        m = (m0, m1, m2, m3, m4)

        lo = s * ROWS_PER_SUB
        pltpu.sync_copy(z_hbm.at[pl.ds(lo, ROWS_PER_SUB)],
                        agg_sh.at[pl.ds(lo, ROWS_PER_SUB)])
        plsc.subcore_barrier()

        @pl.loop(0, ACH, step=AKB)
        def _(blk):
            pltpu.sync_copy(rix_hbm.at[wid, pl.ds(blk, AKB)], rix_v)
            pltpu.sync_copy(dst_hbm.at[wid, pl.ds(blk, AKB)], dst_v)

            g = [None] * NBUF
            sc = [None] * NBUF
            for ch in range(AKB + LEAD):
                if ch < AKB:
                    b = ch % NBUF
                    if sc[b] is not None:
                        sc[b].wait()
                        sc[b] = None
                    g[b] = pltpu.async_copy(
                        t_hbm.at[rix_v.at[ch]], m[b], semg)
                if ch >= LEAD:
                    b = (ch - LEAD) % NBUF
                    g[b].wait()
                    sc[b] = pltpu.async_copy(
                        m[b], agg_sh.at[dst_v.at[ch - LEAD]], sems, add=True)
            for b in range(NBUF):
                if sc[b] is not None:
                    sc[b].wait()

        plsc.subcore_barrier()
        pltpu.sync_copy(agg_sh.at[pl.ds(lo, ROWS_PER_SUB)],
                        p_hbm.at[c, pl.ds(lo, ROWS_PER_SUB)])

    return k(t, rixw, dstw, zagg)


def _sc_degree(dstw, ones, zdeg):
    @functools.partial(
        pl.kernel,
        out_type=jax.ShapeDtypeStruct((NC, N_PAD, D), jnp.float32),
        mesh=plsc.VectorSubcoreMesh(**_MESH),
        scratch_types=[
            pltpu.VMEM((CH, CHUNK), jnp.int32),
            pltpu.VMEM((CHUNK, D), jnp.float32),
            pltpu.VMEM_SHARED((N_PAD, D), jnp.float32),
            pltpu.SemaphoreType.DMA,
        ],
    )
    def k(dst_hbm, ones_hbm, zd_hbm, degp_hbm, dst_v, ones_v, deg_sh, sems):
        c = lax.axis_index("c")
        s = lax.axis_index("s")
        wid = s * NC + c

        pltpu.sync_copy(ones_hbm, ones_v)
        lo = s * ROWS_PER_SUB
        pltpu.sync_copy(zd_hbm.at[pl.ds(lo, ROWS_PER_SUB)],
                        deg_sh.at[pl.ds(lo, ROWS_PER_SUB)])
        plsc.subcore_barrier()

        pltpu.sync_copy(dst_hbm.at[wid], dst_v)
        descs = [pltpu.async_copy(
                     ones_v, deg_sh.at[dst_v.at[ch]], sems, add=True)
                 for ch in range(CH)]
        for d in descs:
            d.wait()

        plsc.subcore_barrier()
        pltpu.sync_copy(deg_sh.at[pl.ds(lo, ROWS_PER_SUB)],
                        degp_hbm.at[c, pl.ds(lo, ROWS_PER_SUB)])

    return k(dstw, ones, zdeg)


def _sc_gather(h3, idx):
    idxw = idx.reshape(NW, GB)

    @functools.partial(
        pl.kernel,
        out_type=jax.ShapeDtypeStruct((2 * B, OUT_PAD), jnp.float32),
        mesh=plsc.VectorSubcoreMesh(**_MESH),
        scratch_types=[
            pltpu.VMEM((GB,), jnp.int32),
            pltpu.VMEM((GB, OUT_PAD), jnp.float32),
            pltpu.SemaphoreType.DMA,
        ],
    )
    def k(h_hbm, i_hbm, o_hbm, idx_v, rows_v, sem):
        c = lax.axis_index("c")
        s = lax.axis_index("s")
        wid = s * NC + c
        pltpu.sync_copy(i_hbm.at[wid], idx_v)
        pltpu.async_copy(h_hbm.at[idx_v], rows_v, sem).wait()
        pltpu.sync_copy(rows_v, o_hbm.at[pl.ds(wid * GB, GB)])

    return k(h3, idxw)


def kernel(feature, W1, Ws1, W2, Ws2, W3, Ws3,
           edge_index, edge_type, head_idx, tail_idx):
    src = edge_index[0]
    dst = edge_index[1]
    pad = E_PAD - E
    srcf = jnp.concatenate([src, jnp.zeros((pad,), jnp.int32)])
    dstf = jnp.concatenate([dst, jnp.full((pad,), N, jnp.int32)])
    etf = jnp.concatenate([edge_type, jnp.zeros((pad,), jnp.int32)])
    dstw = dstf.reshape(NW, CH, CHUNK)
    dsta = dstf.reshape(NW, ACH, AC)
    rixw = _rix(srcf, etf).reshape(NW, ACH, AC)

    zagg = jnp.zeros((N_PAD, D), jnp.float32)
    ones = jnp.ones((CHUNK, D), jnp.float32)

    w3p = jnp.pad(W3, ((0, 0), (0, 0), (0, OUT_PAD - OUT)))
    ws3p = jnp.pad(Ws3, ((0, 0), (0, OUT_PAD - OUT)))

    degp = _sc_degree(dstw, ones, zagg)

    t1 = _transform(feature, W1, D)
    p1 = _sc_aggregate(t1, rixw, dsta, zagg)
    h1 = _combine(p1, degp, feature, Ws1, True, D)

    t2 = _transform(h1, W2, D)
    p2 = _sc_aggregate(t2, rixw, dsta, zagg)
    h2 = _combine(p2, degp, h1, Ws2, True, D)

    t3 = _transform(h2, w3p, OUT_PAD)
    p3 = _sc_aggregate(t3, rixw, dsta, zagg)
    h3 = _combine(p3, degp, h2, ws3p, False, OUT_PAD)

    ht = _sc_gather(h3, jnp.concatenate([head_idx, tail_idx]))
    return ht[:B, :OUT], ht[B:, :OUT]


# speedup vs baseline: 1.2765x; 1.0251x over previous
import functools

import jax
import jax.numpy as jnp
from jax import lax
from jax.experimental import pallas as pl
from jax.experimental.pallas import tpu as pltpu
from jax.experimental.pallas import tpu_sc as plsc

N = 10000
D = 128
R = 4
OUT = 100
OUT_PAD = 128
B = 1024
E = 320000

NC = 2
NS = 16
NW = NC * NS
CHUNK = 128
CH = 80
AC = 64
ACH = 160
AKB = 32
NBUF = 5
LEAD = 3
E_PAD = NW * CH * CHUNK
N_PAD = 10112
ROWS_PER_SUB = N_PAD // NS
NBLK = 5
BN = N // NBLK
GB = 2 * B // NW


def _transform_body(h_ref, w_ref, o_ref):
    o_ref[...] = jnp.dot(h_ref[...], w_ref[0],
                         preferred_element_type=jnp.float32)


def _transform(h, w, dout):
    return pl.pallas_call(
        _transform_body,
        grid=(R, NBLK),
        in_specs=[
            pl.BlockSpec((BN, D), lambda r, i: (i, 0)),
            pl.BlockSpec((1, D, dout), lambda r, i: (r, 0, 0)),
        ],
        out_specs=pl.BlockSpec((BN, dout), lambda r, i: (r * NBLK + i, 0)),
        out_shape=jax.ShapeDtypeStruct((R * N, dout), jnp.float32),
    )(h, w)


def _combine_body(relu, p_ref, degp_ref, h_ref, ws_ref, o_ref):
    deg = degp_ref[0, :, 0:1] + degp_ref[1, :, 0:1]
    agg = (p_ref[0] + p_ref[1]) / jnp.maximum(deg, 1.0)
    out = agg + jnp.dot(h_ref[...], ws_ref[...],
                        preferred_element_type=jnp.float32)
    if relu:
        out = jnp.maximum(out, 0.0)
    o_ref[...] = out


def _combine(p, degp, h, ws, relu, dout):
    return pl.pallas_call(
        functools.partial(_combine_body, relu),
        grid=(NBLK,),
        in_specs=[
            pl.BlockSpec((2, BN, dout), lambda i: (0, i, 0)),
            pl.BlockSpec((2, BN, D), lambda i: (0, i, 0)),
            pl.BlockSpec((BN, D), lambda i: (i, 0)),
            pl.BlockSpec((D, dout), lambda i: (0, 0)),
        ],
        out_specs=pl.BlockSpec((BN, dout), lambda i: (i, 0)),
        out_shape=jax.ShapeDtypeStruct((N, dout), jnp.float32),
    )(p, degp, h, ws)


def _rix_body(s_ref, e_ref, o_ref):
    o_ref[...] = e_ref[...] * N + s_ref[...]


def _rix(srcf, etf):
    rows = E_PAD // 128
    return pl.pallas_call(
        _rix_body,
        grid=(8,),
        in_specs=[pl.BlockSpec((rows // 8, 128), lambda i: (i, 0))] * 2,
        out_specs=pl.BlockSpec((rows // 8, 128), lambda i: (i, 0)),
        out_shape=jax.ShapeDtypeStruct((rows, 128), jnp.int32),
    )(srcf.reshape(rows, 128), etf.reshape(rows, 128))


_MESH = dict(core_axis_name="c", subcore_axis_name="s")


def _sc_aggregate(t, rixw, dstw, zagg):
    @functools.partial(
        pl.kernel,
        out_type=jax.ShapeDtypeStruct((NC, N_PAD, D), jnp.float32),
        mesh=plsc.VectorSubcoreMesh(**_MESH),
        scratch_types=[
            pltpu.VMEM((AKB, AC), jnp.int32),
            pltpu.VMEM((AKB, AC), jnp.int32),
            pltpu.VMEM((AC, D), jnp.float32),
            pltpu.VMEM((AC, D), jnp.float32),
            pltpu.VMEM((AC, D), jnp.float32),
            pltpu.VMEM((AC, D), jnp.float32),
            pltpu.VMEM((AC, D), jnp.float32),
            pltpu.VMEM_SHARED((N_PAD, D), jnp.float32),
            pltpu.SemaphoreType.DMA,
            pltpu.SemaphoreType.DMA,
        ],
    )
    def k(t_hbm, rix_hbm, dst_hbm, z_hbm, p_hbm,
          rix_v, dst_v, m0, m1, m2, m3, m4, agg_sh, semg, sems):
        c = lax.axis_index("c")
        s = lax.axis_index("s")
        wid = s * NC + c
        m = (m0, m1, m2, m3, m4)

        lo = s * ROWS_PER_SUB
        pltpu.sync_copy(z_hbm.at[pl.ds(lo, ROWS_PER_SUB)],
                        agg_sh.at[pl.ds(lo, ROWS_PER_SUB)])
        plsc.subcore_barrier()

        @pl.loop(0, ACH, step=AKB)
        def _(blk):
            pltpu.sync_copy(rix_hbm.at[wid, pl.ds(blk, AKB)], rix_v)
            pltpu.sync_copy(dst_hbm.at[wid, pl.ds(blk, AKB)], dst_v)

            g = [None] * NBUF
            sc = [None] * NBUF
            for ch in range(AKB + LEAD):
                if ch < AKB:
                    b = ch % NBUF
                    if sc[b] is not None:
                        sc[b].wait()
                        sc[b] = None
                    g[b] = pltpu.async_copy(
                        t_hbm.at[rix_v.at[ch]], m[b], semg)
                if ch >= LEAD:
                    b = (ch - LEAD) % NBUF
                    g[b].wait()
                    sc[b] = pltpu.async_copy(
                        m[b], agg_sh.at[dst_v.at[ch - LEAD]], sems, add=True)
            for b in range(NBUF):
                if sc[b] is not None:
                    sc[b].wait()

        plsc.subcore_barrier()
        pltpu.sync_copy(agg_sh.at[pl.ds(lo, ROWS_PER_SUB)],
                        p_hbm.at[c, pl.ds(lo, ROWS_PER_SUB)])

    return k(t, rixw, dstw, zagg)


def _sc_degree(dstw, ones, zdeg):
    @functools.partial(
        pl.kernel,
        out_type=jax.ShapeDtypeStruct((NC, N_PAD, D), jnp.float32),
        mesh=plsc.VectorSubcoreMesh(**_MESH),
        scratch_types=[
            pltpu.VMEM((CH, CHUNK), jnp.int32),
            pltpu.VMEM((CHUNK, D), jnp.float32),
            pltpu.VMEM_SHARED((N_PAD, D), jnp.float32),
            pltpu.SemaphoreType.DMA,
        ],
    )
    def k(dst_hbm, ones_hbm, zd_hbm, degp_hbm, dst_v, ones_v, deg_sh, sems):
        c = lax.axis_index("c")
        s = lax.axis_index("s")
        wid = s * NC + c

        pltpu.sync_copy(ones_hbm, ones_v)
        lo = s * ROWS_PER_SUB
        pltpu.sync_copy(zd_hbm.at[pl.ds(lo, ROWS_PER_SUB)],
                        deg_sh.at[pl.ds(lo, ROWS_PER_SUB)])
        plsc.subcore_barrier()

        pltpu.sync_copy(dst_hbm.at[wid], dst_v)
        descs = [pltpu.async_copy(
                     ones_v, deg_sh.at[dst_v.at[ch]], sems, add=True)
                 for ch in range(CH)]
        for d in descs:
            d.wait()

        plsc.subcore_barrier()
        pltpu.sync_copy(deg_sh.at[pl.ds(lo, ROWS_PER_SUB)],
                        degp_hbm.at[c, pl.ds(lo, ROWS_PER_SUB)])

    return k(dstw, ones, zdeg)


def _sc_gather(h3, idx):
    idxw = idx.reshape(NW, GB)

    @functools.partial(
        pl.kernel,
        out_type=jax.ShapeDtypeStruct((2 * B, OUT_PAD), jnp.float32),
        mesh=plsc.VectorSubcoreMesh(**_MESH),
        scratch_types=[
            pltpu.VMEM((GB,), jnp.int32),
            pltpu.VMEM((GB, OUT_PAD), jnp.float32),
            pltpu.SemaphoreType.DMA,
        ],
    )
    def k(h_hbm, i_hbm, o_hbm, idx_v, rows_v, sem):
        c = lax.axis_index("c")
        s = lax.axis_index("s")
        wid = s * NC + c
        pltpu.sync_copy(i_hbm.at[wid], idx_v)
        pltpu.async_copy(h_hbm.at[idx_v], rows_v, sem).wait()
        pltpu.sync_copy(rows_v, o_hbm.at[pl.ds(wid * GB, GB)])

    return k(h3, idxw)


def kernel(feature, W1, Ws1, W2, Ws2, W3, Ws3,
           edge_index, edge_type, head_idx, tail_idx):
    src = edge_index[0]
    dst = edge_index[1]
    pad = E_PAD - E
    srcf = jnp.concatenate([src, jnp.zeros((pad,), jnp.int32)])
    dstf = jnp.concatenate([dst, jnp.full((pad,), N, jnp.int32)])
    etf = jnp.concatenate([edge_type, jnp.zeros((pad,), jnp.int32)])
    dstw = dstf.reshape(NW, CH, CHUNK)
    dsta = dstf.reshape(NW, ACH, AC)
    rixw = _rix(srcf, etf).reshape(NW, ACH, AC)

    zagg = jnp.zeros((N_PAD, D), jnp.float32)
    ones = jnp.ones((CHUNK, D), jnp.float32)

    w3p = jnp.pad(W3, ((0, 0), (0, 0), (0, OUT_PAD - OUT)))
    ws3p = jnp.pad(Ws3, ((0, 0), (0, OUT_PAD - OUT)))

    degp = _sc_degree(dstw, ones, zagg)

    t1 = _transform(feature, W1, D)
    p1 = _sc_aggregate(t1, rixw, dsta, zagg)
    h1 = _combine(p1, degp, feature, Ws1, True, D)

    t2 = _transform(h1, W2, D)
    p2 = _sc_aggregate(t2, rixw, dsta, zagg)
    h2 = _combine(p2, degp, h1, Ws2, True, D)

    t3 = _transform(h2, w3p, OUT_PAD)
    p3 = _sc_aggregate(t3, rixw, dsta, zagg)
    h3 = _combine(p3, degp, h2, ws3p, False, OUT_PAD)

    ht = _sc_gather(h3, jnp.concatenate([head_idx, tail_idx]))
    return ht[:B, :OUT], ht[B:, :OUT]


# speedup vs baseline: 1.2897x; 1.0104x over previous
import functools

import jax
import jax.numpy as jnp
from jax import lax
from jax.experimental import pallas as pl
from jax.experimental.pallas import tpu as pltpu
from jax.experimental.pallas import tpu_sc as plsc

N = 10000
D = 128
R = 4
OUT = 100
OUT_PAD = 128
B = 1024
E = 320000

NC = 2
NS = 16
NW = NC * NS
CHUNK = 128
CH = 80
AC = 64
ACH = 160
AKB = 32
NBUF = 5
LEAD = 3
E_PAD = NW * CH * CHUNK
N_PAD = 10112
ROWS_PER_SUB = N_PAD // NS
NBLK = 2
BN = N // NBLK
GB = 2 * B // NW


def _transform_body(h_ref, w_ref, o_ref):
    o_ref[...] = jnp.dot(h_ref[...], w_ref[0],
                         preferred_element_type=jnp.float32)


def _transform(h, w, dout):
    return pl.pallas_call(
        _transform_body,
        grid=(R, NBLK),
        in_specs=[
            pl.BlockSpec((BN, D), lambda r, i: (i, 0)),
            pl.BlockSpec((1, D, dout), lambda r, i: (r, 0, 0)),
        ],
        out_specs=pl.BlockSpec((BN, dout), lambda r, i: (r * NBLK + i, 0)),
        out_shape=jax.ShapeDtypeStruct((R * N, dout), jnp.float32),
    )(h, w)


def _combine_body(relu, p_ref, degp_ref, h_ref, ws_ref, o_ref):
    deg = degp_ref[0, :, 0:1] + degp_ref[1, :, 0:1]
    agg = (p_ref[0] + p_ref[1]) / jnp.maximum(deg, 1.0)
    out = agg + jnp.dot(h_ref[...], ws_ref[...],
                        preferred_element_type=jnp.float32)
    if relu:
        out = jnp.maximum(out, 0.0)
    o_ref[...] = out


def _combine(p, degp, h, ws, relu, dout):
    return pl.pallas_call(
        functools.partial(_combine_body, relu),
        grid=(NBLK,),
        in_specs=[
            pl.BlockSpec((2, BN, dout), lambda i: (0, i, 0)),
            pl.BlockSpec((2, BN, D), lambda i: (0, i, 0)),
            pl.BlockSpec((BN, D), lambda i: (i, 0)),
            pl.BlockSpec((D, dout), lambda i: (0, 0)),
        ],
        out_specs=pl.BlockSpec((BN, dout), lambda i: (i, 0)),
        out_shape=jax.ShapeDtypeStruct((N, dout), jnp.float32),
    )(p, degp, h, ws)


def _rix_body(s_ref, e_ref, o_ref):
    o_ref[...] = e_ref[...] * N + s_ref[...]


def _rix(srcf, etf):
    rows = E_PAD // 128
    return pl.pallas_call(
        _rix_body,
        grid=(8,),
        in_specs=[pl.BlockSpec((rows // 8, 128), lambda i: (i, 0))] * 2,
        out_specs=pl.BlockSpec((rows // 8, 128), lambda i: (i, 0)),
        out_shape=jax.ShapeDtypeStruct((rows, 128), jnp.int32),
    )(srcf.reshape(rows, 128), etf.reshape(rows, 128))


_MESH = dict(core_axis_name="c", subcore_axis_name="s")


def _sc_aggregate(t, rixw, dstw, zagg):
    @functools.partial(
        pl.kernel,
        out_type=jax.ShapeDtypeStruct((NC, N_PAD, D), jnp.float32),
        mesh=plsc.VectorSubcoreMesh(**_MESH),
        scratch_types=[
            pltpu.VMEM((AKB, AC), jnp.int32),
            pltpu.VMEM((AKB, AC), jnp.int32),
            pltpu.VMEM((AC, D), jnp.float32),
            pltpu.VMEM((AC, D), jnp.float32),
            pltpu.VMEM((AC, D), jnp.float32),
            pltpu.VMEM((AC, D), jnp.float32),
            pltpu.VMEM((AC, D), jnp.float32),
            pltpu.VMEM_SHARED((N_PAD, D), jnp.float32),
            pltpu.SemaphoreType.DMA,
            pltpu.SemaphoreType.DMA,
        ],
    )
    def k(t_hbm, rix_hbm, dst_hbm, z_hbm, p_hbm,
          rix_v, dst_v, m0, m1, m2, m3, m4, agg_sh, semg, sems):
        c = lax.axis_index("c")
        s = lax.axis_index("s")
        wid = s * NC + c
        m = (m0, m1, m2, m3, m4)

        lo = s * ROWS_PER_SUB
        pltpu.sync_copy(z_hbm.at[pl.ds(lo, ROWS_PER_SUB)],
                        agg_sh.at[pl.ds(lo, ROWS_PER_SUB)])
        plsc.subcore_barrier()

        @pl.loop(0, ACH, step=AKB)
        def _(blk):
            pltpu.sync_copy(rix_hbm.at[wid, pl.ds(blk, AKB)], rix_v)
            pltpu.sync_copy(dst_hbm.at[wid, pl.ds(blk, AKB)], dst_v)

            g = [None] * NBUF
            sc = [None] * NBUF
            for ch in range(AKB + LEAD):
                if ch < AKB:
                    b = ch % NBUF
                    if sc[b] is not None:
                        sc[b].wait()
                        sc[b] = None
                    g[b] = pltpu.async_copy(
                        t_hbm.at[rix_v.at[ch]], m[b], semg)
                if ch >= LEAD:
                    b = (ch - LEAD) % NBUF
                    g[b].wait()
                    sc[b] = pltpu.async_copy(
                        m[b], agg_sh.at[dst_v.at[ch - LEAD]], sems, add=True)
            for b in range(NBUF):
                if sc[b] is not None:
                    sc[b].wait()

        plsc.subcore_barrier()
        pltpu.sync_copy(agg_sh.at[pl.ds(lo, ROWS_PER_SUB)],
                        p_hbm.at[c, pl.ds(lo, ROWS_PER_SUB)])

    return k(t, rixw, dstw, zagg)


def _sc_degree(dstw, ones, zdeg):
    @functools.partial(
        pl.kernel,
        out_type=jax.ShapeDtypeStruct((NC, N_PAD, D), jnp.float32),
        mesh=plsc.VectorSubcoreMesh(**_MESH),
        scratch_types=[
            pltpu.VMEM((CH, CHUNK), jnp.int32),
            pltpu.VMEM((CHUNK, D), jnp.float32),
            pltpu.VMEM_SHARED((N_PAD, D), jnp.float32),
            pltpu.SemaphoreType.DMA,
        ],
    )
    def k(dst_hbm, ones_hbm, zd_hbm, degp_hbm, dst_v, ones_v, deg_sh, sems):
        c = lax.axis_index("c")
        s = lax.axis_index("s")
        wid = s * NC + c

        pltpu.sync_copy(ones_hbm, ones_v)
        lo = s * ROWS_PER_SUB
        pltpu.sync_copy(zd_hbm.at[pl.ds(lo, ROWS_PER_SUB)],
                        deg_sh.at[pl.ds(lo, ROWS_PER_SUB)])
        plsc.subcore_barrier()

        pltpu.sync_copy(dst_hbm.at[wid], dst_v)
        descs = [pltpu.async_copy(
                     ones_v, deg_sh.at[dst_v.at[ch]], sems, add=True)
                 for ch in range(CH)]
        for d in descs:
            d.wait()

        plsc.subcore_barrier()
        pltpu.sync_copy(deg_sh.at[pl.ds(lo, ROWS_PER_SUB)],
                        degp_hbm.at[c, pl.ds(lo, ROWS_PER_SUB)])

    return k(dstw, ones, zdeg)


def _sc_gather(h3, idx):
    idxw = idx.reshape(NW, GB)

    @functools.partial(
        pl.kernel,
        out_type=jax.ShapeDtypeStruct((2 * B, OUT_PAD), jnp.float32),
        mesh=plsc.VectorSubcoreMesh(**_MESH),
        scratch_types=[
            pltpu.VMEM((GB,), jnp.int32),
            pltpu.VMEM((GB, OUT_PAD), jnp.float32),
            pltpu.SemaphoreType.DMA,
        ],
    )
    def k(h_hbm, i_hbm, o_hbm, idx_v, rows_v, sem):
        c = lax.axis_index("c")
        s = lax.axis_index("s")
        wid = s * NC + c
        pltpu.sync_copy(i_hbm.at[wid], idx_v)
        pltpu.async_copy(h_hbm.at[idx_v], rows_v, sem).wait()
        pltpu.sync_copy(rows_v, o_hbm.at[pl.ds(wid * GB, GB)])

    return k(h3, idxw)


def kernel(feature, W1, Ws1, W2, Ws2, W3, Ws3,
           edge_index, edge_type, head_idx, tail_idx):
    src = edge_index[0]
    dst = edge_index[1]
    pad = E_PAD - E
    srcf = jnp.concatenate([src, jnp.zeros((pad,), jnp.int32)])
    dstf = jnp.concatenate([dst, jnp.full((pad,), N, jnp.int32)])
    etf = jnp.concatenate([edge_type, jnp.zeros((pad,), jnp.int32)])
    dstw = dstf.reshape(NW, CH, CHUNK)
    dsta = dstf.reshape(NW, ACH, AC)
    rixw = _rix(srcf, etf).reshape(NW, ACH, AC)

    zagg = jnp.zeros((N_PAD, D), jnp.float32)
    ones = jnp.ones((CHUNK, D), jnp.float32)

    w3p = jnp.pad(W3, ((0, 0), (0, 0), (0, OUT_PAD - OUT)))
    ws3p = jnp.pad(Ws3, ((0, 0), (0, OUT_PAD - OUT)))

    degp = _sc_degree(dstw, ones, zagg)

    t1 = _transform(feature, W1, D)
    p1 = _sc_aggregate(t1, rixw, dsta, zagg)
    h1 = _combine(p1, degp, feature, Ws1, True, D)

    t2 = _transform(h1, W2, D)
    p2 = _sc_aggregate(t2, rixw, dsta, zagg)
    h2 = _combine(p2, degp, h1, Ws2, True, D)

    t3 = _transform(h2, w3p, OUT_PAD)
    p3 = _sc_aggregate(t3, rixw, dsta, zagg)
    h3 = _combine(p3, degp, h2, ws3p, False, OUT_PAD)

    ht = _sc_gather(h3, jnp.concatenate([head_idx, tail_idx]))
    return ht[:B, :OUT], ht[B:, :OUT]
